# Initial kernel scaffold; baseline (speedup 1.0000x reference)
import jax
import jax.numpy as jnp
from jax.experimental import pallas as pl


def kernel(node_embeddings, node_sizes, W_inner, b_inner, W_outer, b_outer):
    raise NotImplementedError("write your pallas kernel here")


# speedup vs baseline: 53.0805x; 53.0805x over previous
import functools

import jax
import jax.numpy as jnp
from jax import lax
from jax.experimental import pallas as pl
from jax.experimental.pallas import tpu as pltpu

_RC = 512


def _sum_readout_kern(nc_ref, x_hbm, starts_ref, ends_ref, wi_ref, bi_ref,
                      wo_ref, bo_ref, out_ref, xbuf, acc_ref, sem):
    Bp = starts_ref.shape[0]
    RC = xbuf.shape[1]
    nchunks = nc_ref[0]

    starts = starts_ref[...]
    ends = ends_ref[...]
    acc_ref[...] = jnp.zeros_like(acc_ref)

    def cp(c, slot):
        return pltpu.make_async_copy(
            x_hbm.at[pl.ds(c * RC, RC), :], xbuf.at[slot], sem.at[slot])

    @pl.when(nchunks > 0)
    def _():
        cp(0, 0).start()

    def body(c, carry):
        slot = lax.rem(c, 2)

        @pl.when(c + 1 < nchunks)
        def _():
            cp(c + 1, 1 - slot).start()

        cp(c, slot).wait()
        x = xbuf[slot]
        r = lax.broadcasted_iota(jnp.int32, (Bp, RC), 1) + c * RC
        m = jnp.where((r >= starts) & (r < ends), 1.0, 0.0)
        acc_ref[...] += lax.dot_general(
            m, x, (((1,), (0,)), ((), ())), preferred_element_type=jnp.float32)
        return carry

    lax.fori_loop(0, nchunks, body, 0)

    agg = acc_ref[...]
    h = lax.dot_general(agg, wi_ref[...], (((1,), (1,)), ((), ())),
                        preferred_element_type=jnp.float32) + bi_ref[...]
    sp = jnp.maximum(h, 0.0) + jnp.log1p(jnp.exp(-jnp.abs(h)))
    h = h * jnp.tanh(sp)
    out_ref[...] = lax.dot_general(
        h, wo_ref[...], (((1,), (1,)), ((), ())),
        preferred_element_type=jnp.float32) + bo_ref[...]


@functools.partial(jax.jit, static_argnames=("interpret",))
def _sum_readout(node_embeddings, node_sizes, W_inner, b_inner, W_outer,
                 b_outer, interpret=False):
    N, d_in = node_embeddings.shape
    B = node_sizes.shape[0]
    d_out = W_outer.shape[0]
    Bp = ((B + 127) // 128) * 128

    ends_i = jnp.cumsum(node_sizes.astype(jnp.int32))
    starts_i = ends_i - node_sizes.astype(jnp.int32)
    n_rows = ends_i[-1]
    nc = lax.div(n_rows + (_RC - 1), _RC).reshape(1).astype(jnp.int32)
    pad = jnp.full((Bp - B,), n_rows, jnp.int32)
    ends_p = jnp.concatenate([ends_i, pad]).reshape(Bp, 1)
    starts_p = jnp.concatenate([starts_i, pad]).reshape(Bp, 1)

    out = pl.pallas_call(
        _sum_readout_kern,
        out_shape=jax.ShapeDtypeStruct((Bp, d_out), jnp.float32),
        in_specs=[
            pl.BlockSpec(memory_space=pltpu.SMEM),
            pl.BlockSpec(memory_space=pl.ANY),
            pl.BlockSpec(memory_space=pltpu.VMEM),
            pl.BlockSpec(memory_space=pltpu.VMEM),
            pl.BlockSpec(memory_space=pltpu.VMEM),
            pl.BlockSpec(memory_space=pltpu.VMEM),
            pl.BlockSpec(memory_space=pltpu.VMEM),
            pl.BlockSpec(memory_space=pltpu.VMEM),
        ],
        out_specs=pl.BlockSpec(memory_space=pltpu.VMEM),
        scratch_shapes=[
            pltpu.VMEM((2, _RC, d_in), jnp.float32),
            pltpu.VMEM((Bp, d_in), jnp.float32),
            pltpu.SemaphoreType.DMA((2,)),
        ],
        interpret=interpret,
    )(nc, node_embeddings, starts_p, ends_p, W_inner,
      b_inner.reshape(1, -1), W_outer, b_outer.reshape(1, -1))
    return out[:B]


def kernel(node_embeddings, node_sizes, W_inner, b_inner, W_outer, b_outer):
    return _sum_readout(node_embeddings, node_sizes, W_inner, b_inner,
                        W_outer, b_outer)
